# baseline (speedup 1.0000x reference)
import functools

import jax
import jax.numpy as jnp
from jax import lax
from jax.experimental import pallas as pl
from jax.experimental.pallas import tpu as pltpu
import jax.experimental.pallas.tpu_sc as plsc

N = 10000
E = 320000
D_IN = 128
D_HID = 128
D_LAT = 32
D_OUT = 128

NC = 2
NS = 16
NW = NC * NS
N_PAD = 10240
ROWS_PER_TILE = N_PAD // NS
C = 128
E_PAD = 327680
EW = E_PAD // NW
NCHUNK = EW // C
NHALF = 2
NCHUNK_H = NCHUNK // NHALF
PAD_DST = N - 1 + (N_PAD - N) // 2

_MESH = dict(core_axis_name="c", subcore_axis_name="s")


def _worker_id():
    return lax.axis_index("s") * NC + lax.axis_index("c")


@functools.partial(
    pl.kernel,
    out_type=jax.ShapeDtypeStruct((NC, N_PAD), jnp.float32),
    mesh=plsc.VectorSubcoreMesh(**_MESH),
    scratch_types=[
        pltpu.VMEM((NCHUNK, C), jnp.int32),
        pltpu.VMEM((C,), jnp.float32),
        pltpu.VMEM_SHARED((N_PAD,), jnp.float32),
    ],
)
def _sc_deg(dst3d, zeros1d, out, idx_v, ones_v, acc):
    cid = lax.axis_index("c")
    sid = lax.axis_index("s")
    w = _worker_id()
    pltpu.sync_copy(zeros1d, acc.at[pl.ds(sid * ROWS_PER_TILE, ROWS_PER_TILE)])
    for i in range(C // 16):
        ones_v[pl.ds(i * 16, 16)] = jnp.ones((16,), jnp.float32)
    pltpu.sync_copy(dst3d.at[w], idx_v)
    plsc.subcore_barrier()

    def body(j, carry):
        pltpu.sync_copy(ones_v, acc.at[idx_v.at[j]], add=True)
        return carry

    lax.fori_loop(0, NCHUNK, body, 0)
    plsc.subcore_barrier()
    pltpu.sync_copy(
        acc.at[pl.ds(sid * ROWS_PER_TILE, ROWS_PER_TILE)],
        out.at[cid].at[pl.ds(sid * ROWS_PER_TILE, ROWS_PER_TILE)],
    )


def _make_sc_segsum(D):
    @functools.partial(
        pl.kernel,
        out_type=jax.ShapeDtypeStruct((NC, N_PAD, D), jnp.float32),
        mesh=plsc.VectorSubcoreMesh(**_MESH),
        scratch_types=[
            pltpu.VMEM((NCHUNK_H, C), jnp.int32),
            pltpu.VMEM((NCHUNK_H, C), jnp.int32),
            pltpu.VMEM((2, C, D), jnp.float32),
            pltpu.VMEM_SHARED((N_PAD, D), jnp.float32),
            pltpu.SemaphoreType.DMA,
        ],
    )
    def _sc_segsum(u_hbm, src4d, dst4d, zeros2d, out, src_v, dst_v, rows_v, acc, sem):
        cid = lax.axis_index("c")
        sid = lax.axis_index("s")
        w = _worker_id()
        pltpu.sync_copy(zeros2d, acc.at[pl.ds(sid * ROWS_PER_TILE, ROWS_PER_TILE)])
        plsc.subcore_barrier()

        for half in range(NHALF):
            pltpu.sync_copy(src4d.at[w].at[half], src_v)
            pltpu.sync_copy(dst4d.at[w].at[half], dst_v)
            def body(j, carry):
                pltpu.async_copy(u_hbm.at[src_v.at[j]], rows_v.at[0], sem).wait()
                pltpu.sync_copy(rows_v.at[0], acc.at[dst_v.at[j]], add=True)
                return carry

            lax.fori_loop(0, NCHUNK_H, body, 0)
        plsc.subcore_barrier()
        pltpu.sync_copy(
            acc.at[pl.ds(sid * ROWS_PER_TILE, ROWS_PER_TILE)],
            out.at[cid].at[pl.ds(sid * ROWS_PER_TILE, ROWS_PER_TILE)],
        )

    return _sc_segsum


_sc_segsum_128 = _make_sc_segsum(D_HID)

_BM = 1000


def _tc1_body(x_ref, w1_ref, degp_ref, u1_ref, dinv_ref):
    deg = degp_ref[:, 0] + degp_ref[:, 1] + 1.0
    dinv = lax.rsqrt(deg)
    xw = jnp.dot(x_ref[...], w1_ref[...], preferred_element_type=jnp.float32)
    u1_ref[...] = dinv[:, None] * xw
    dinv_ref[...] = dinv[:, None]


def _tc1(x, w1, degp):
    return pl.pallas_call(
        _tc1_body,
        grid=(N // _BM,),
        in_specs=[
            pl.BlockSpec((_BM, D_IN), lambda i: (i, 0)),
            pl.BlockSpec((D_IN, D_HID), lambda i: (0, 0)),
            pl.BlockSpec((_BM, NC), lambda i: (i, 0)),
        ],
        out_specs=[
            pl.BlockSpec((_BM, D_HID), lambda i: (i, 0)),
            pl.BlockSpec((_BM, 1), lambda i: (i, 0)),
        ],
        out_shape=[
            jax.ShapeDtypeStruct((N, D_HID), jnp.float32),
            jax.ShapeDtypeStruct((N, 1), jnp.float32),
        ],
    )(x, w1, degp)


def _tc2_body(u1_ref, aggp_ref, dinv_ref, b1_ref, w2_ref, u2_ref):
    agg = aggp_ref[0] + aggp_ref[1]
    dinv = dinv_ref[...]
    h = jnp.maximum(dinv * (agg + u1_ref[...]) + b1_ref[...], 0.0)
    v = jnp.dot(h, w2_ref[...], preferred_element_type=jnp.float32)
    u2_ref[...] = dinv * v


def _tc2(u1, aggp, dinv, b1, w2):
    return pl.pallas_call(
        _tc2_body,
        grid=(N // _BM,),
        in_specs=[
            pl.BlockSpec((_BM, D_HID), lambda i: (i, 0)),
            pl.BlockSpec((NC, _BM, D_HID), lambda i: (0, i, 0)),
            pl.BlockSpec((_BM, 1), lambda i: (i, 0)),
            pl.BlockSpec((1, D_HID), lambda i: (0, 0)),
            pl.BlockSpec((D_HID, D_HID), lambda i: (0, 0)),
        ],
        out_specs=pl.BlockSpec((_BM, D_HID), lambda i: (i, 0)),
        out_shape=jax.ShapeDtypeStruct((N, D_HID), jnp.float32),
    )(u1, aggp, dinv, b1, w2)


def _tc3_body(u2_ref, aggp_ref, dinv_ref, bmu_ref, bls_ref, decw_ref, mask_ref,
              decb_ref, mu_ref, ls_ref, expr_ref):
    out2 = dinv_ref[...] * (aggp_ref[0] + aggp_ref[1] + u2_ref[...])
    mu = out2[:, :D_LAT] + bmu_ref[...]
    ls = out2[:, D_LAT:2 * D_LAT] + bls_ref[...]
    mu_ref[...] = mu
    ls_ref[...] = ls
    t = jnp.dot(mu, decw_ref[...] * mask_ref[...],
                preferred_element_type=jnp.float32) + decb_ref[...]
    t = t - jnp.max(t, axis=-1, keepdims=True)
    et = jnp.exp(t)
    expr_ref[...] = et / jnp.sum(et, axis=-1, keepdims=True)


def _tc3(u2, aggp, dinv, bmu, bls, decw, mask, decb):
    return pl.pallas_call(
        _tc3_body,
        grid=(N // _BM,),
        in_specs=[
            pl.BlockSpec((_BM, D_HID), lambda i: (i, 0)),
            pl.BlockSpec((NC, _BM, D_HID), lambda i: (0, i, 0)),
            pl.BlockSpec((_BM, 1), lambda i: (i, 0)),
            pl.BlockSpec((1, D_LAT), lambda i: (0, 0)),
            pl.BlockSpec((1, D_LAT), lambda i: (0, 0)),
            pl.BlockSpec((D_LAT, D_OUT), lambda i: (0, 0)),
            pl.BlockSpec((D_LAT, D_OUT), lambda i: (0, 0)),
            pl.BlockSpec((1, D_OUT), lambda i: (0, 0)),
        ],
        out_specs=[
            pl.BlockSpec((_BM, D_LAT), lambda i: (i, 0)),
            pl.BlockSpec((_BM, D_LAT), lambda i: (i, 0)),
            pl.BlockSpec((_BM, D_OUT), lambda i: (i, 0)),
        ],
        out_shape=[
            jax.ShapeDtypeStruct((N, D_LAT), jnp.float32),
            jax.ShapeDtypeStruct((N, D_LAT), jnp.float32),
            jax.ShapeDtypeStruct((N, D_OUT), jnp.float32),
        ],
    )(u2, aggp, dinv, bmu, bls, decw, mask, decb)


def _tc4_body(a_ref, b_ref, out_ref):
    out_ref[...] = lax.dot_general(
        a_ref[...], b_ref[...], (((1,), (1,)), ((), ())),
        preferred_element_type=jnp.float32)


_BM4 = 400


def _tc4(z):
    return pl.pallas_call(
        _tc4_body,
        grid=(N // _BM4,),
        in_specs=[
            pl.BlockSpec((_BM4, D_LAT), lambda i: (i, 0)),
            pl.BlockSpec((N, D_LAT), lambda i: (0, 0)),
        ],
        out_specs=pl.BlockSpec((_BM4, N), lambda i: (i, 0)),
        out_shape=jax.ShapeDtypeStruct((N, N), jnp.float32),
    )(z, z)


def kernel(x, edge_index, W1, b1, Wmu, bmu, Wls, bls, decW, decb, dec_mask):
    pad = E_PAD - E
    src_p = jnp.concatenate([edge_index[0], jnp.zeros((pad,), jnp.int32)])
    dst_p = jnp.concatenate([edge_index[1], jnp.full((pad,), PAD_DST, jnp.int32)])
    src2d = src_p.reshape(NW, NHALF, NCHUNK_H, C)
    dst2d = dst_p.reshape(NW, NHALF, NCHUNK_H, C)
    dst3d_deg = dst_p.reshape(NW, NCHUNK, C)
    z1 = jnp.zeros((ROWS_PER_TILE,), jnp.float32)
    z128 = jnp.zeros((ROWS_PER_TILE, D_HID), jnp.float32)

    degp = _sc_deg(dst3d_deg, z1)
    u1, dinv = _tc1(x, W1, degp.T)
    agg1p = _sc_segsum_128(u1, src2d, dst2d, z128)
    w2 = jnp.concatenate(
        [Wmu, Wls, jnp.zeros((D_HID, D_HID - 2 * D_LAT), jnp.float32)], axis=1)
    u2 = _tc2(u1, agg1p, dinv, b1.reshape(1, D_HID), w2)
    agg2p = _sc_segsum_128(u2, src2d, dst2d, z128)
    mu, logstd, expr = _tc3(
        u2, agg2p, dinv, bmu.reshape(1, D_LAT), bls.reshape(1, D_LAT),
        decW, dec_mask, decb.reshape(1, D_OUT))
    adj = _tc4(mu)
    return (adj, expr, mu, logstd)


# speedup vs baseline: 2.8148x; 2.8148x over previous
import functools

import jax
import jax.numpy as jnp
from jax import lax
from jax.experimental import pallas as pl
from jax.experimental.pallas import tpu as pltpu
import jax.experimental.pallas.tpu_sc as plsc

N = 10000
E = 320000
D_IN = 128
D_HID = 128
D_LAT = 32
D_OUT = 128

NC = 2
NS = 16
NW = NC * NS
N_PAD = 10240
ROWS_PER_TILE = N_PAD // NS
C = 80
E_PAD = E
EW = E_PAD // NW
NCHUNK = EW // C
NHALF = 5
NCHUNK_H = NCHUNK // NHALF
PAD_DST = N - 1 + (N_PAD - N) // 2

_MESH = dict(core_axis_name="c", subcore_axis_name="s")


def _worker_id():
    return lax.axis_index("s") * NC + lax.axis_index("c")


@functools.partial(
    pl.kernel,
    out_type=jax.ShapeDtypeStruct((NC, N_PAD), jnp.float32),
    mesh=plsc.VectorSubcoreMesh(**_MESH),
    scratch_types=[
        pltpu.VMEM((NCHUNK, C), jnp.int32),
        pltpu.VMEM((C,), jnp.float32),
        pltpu.VMEM_SHARED((N_PAD,), jnp.float32),
    ],
)
def _sc_deg(dst3d, zeros1d, out, idx_v, ones_v, acc):
    cid = lax.axis_index("c")
    sid = lax.axis_index("s")
    w = _worker_id()
    pltpu.sync_copy(zeros1d, acc.at[pl.ds(sid * ROWS_PER_TILE, ROWS_PER_TILE)])
    for i in range(C // 16):
        ones_v[pl.ds(i * 16, 16)] = jnp.ones((16,), jnp.float32)
    pltpu.sync_copy(dst3d.at[w], idx_v)
    plsc.subcore_barrier()

    def body(j, carry):
        pltpu.sync_copy(ones_v, acc.at[idx_v.at[j]], add=True)
        return carry

    lax.fori_loop(0, NCHUNK, body, 0)
    plsc.subcore_barrier()
    pltpu.sync_copy(
        acc.at[pl.ds(sid * ROWS_PER_TILE, ROWS_PER_TILE)],
        out.at[cid].at[pl.ds(sid * ROWS_PER_TILE, ROWS_PER_TILE)],
    )


def _make_sc_segsum(D):
    @functools.partial(
        pl.kernel,
        out_type=jax.ShapeDtypeStruct((NC, N_PAD, D), jnp.float32),
        mesh=plsc.VectorSubcoreMesh(**_MESH),
        scratch_types=[
            pltpu.VMEM((NCHUNK_H, C), jnp.int32),
            pltpu.VMEM((NCHUNK_H, C), jnp.int32),
            pltpu.VMEM((2, C, D), jnp.float32),
            pltpu.VMEM_SHARED((N_PAD, D), jnp.float32),
            pltpu.SemaphoreType.DMA,
        ],
    )
    def _sc_segsum(u_hbm, src4d, dst4d, zeros2d, out, src_v, dst_v, rows_v, acc, sem):
        cid = lax.axis_index("c")
        sid = lax.axis_index("s")
        w = _worker_id()
        pltpu.sync_copy(zeros2d, acc.at[pl.ds(sid * ROWS_PER_TILE, ROWS_PER_TILE)])
        plsc.subcore_barrier()

        for half in range(NHALF):
            pltpu.sync_copy(src4d.at[w].at[half], src_v)
            pltpu.sync_copy(dst4d.at[w].at[half], dst_v)
            pltpu.async_copy(u_hbm.at[src_v.at[0]], rows_v.at[0], sem)

            def body(j, carry):
                b = lax.rem(j, 2)
                @pl.when(j + 1 < NCHUNK_H)
                def _():
                    pltpu.async_copy(
                        u_hbm.at[src_v.at[j + 1]], rows_v.at[1 - b], sem)
                pltpu.make_async_copy(
                    u_hbm.at[src_v.at[j]], rows_v.at[b], sem).wait()
                pltpu.sync_copy(rows_v.at[b], acc.at[dst_v.at[j]], add=True)
                return carry

            lax.fori_loop(0, NCHUNK_H, body, 0)
        plsc.subcore_barrier()
        pltpu.sync_copy(
            acc.at[pl.ds(sid * ROWS_PER_TILE, ROWS_PER_TILE)],
            out.at[cid].at[pl.ds(sid * ROWS_PER_TILE, ROWS_PER_TILE)],
        )

    return _sc_segsum


_sc_segsum_128 = _make_sc_segsum(D_HID)

_BM = 1000


def _tc1_body(x_ref, w1_ref, degp_ref, u1_ref, dinv_ref):
    deg = degp_ref[:, 0] + degp_ref[:, 1] + 1.0
    dinv = lax.rsqrt(deg)
    xw = jnp.dot(x_ref[...], w1_ref[...], preferred_element_type=jnp.float32)
    u1_ref[...] = dinv[:, None] * xw
    dinv_ref[...] = dinv[:, None]


def _tc1(x, w1, degp):
    return pl.pallas_call(
        _tc1_body,
        grid=(N // _BM,),
        in_specs=[
            pl.BlockSpec((_BM, D_IN), lambda i: (i, 0)),
            pl.BlockSpec((D_IN, D_HID), lambda i: (0, 0)),
            pl.BlockSpec((_BM, NC), lambda i: (i, 0)),
        ],
        out_specs=[
            pl.BlockSpec((_BM, D_HID), lambda i: (i, 0)),
            pl.BlockSpec((_BM, 1), lambda i: (i, 0)),
        ],
        out_shape=[
            jax.ShapeDtypeStruct((N, D_HID), jnp.float32),
            jax.ShapeDtypeStruct((N, 1), jnp.float32),
        ],
    )(x, w1, degp)


def _tc2_body(u1_ref, aggp_ref, dinv_ref, b1_ref, w2_ref, u2_ref):
    agg = aggp_ref[0] + aggp_ref[1]
    dinv = dinv_ref[...]
    h = jnp.maximum(dinv * (agg + u1_ref[...]) + b1_ref[...], 0.0)
    v = jnp.dot(h, w2_ref[...], preferred_element_type=jnp.float32)
    u2_ref[...] = dinv * v


def _tc2(u1, aggp, dinv, b1, w2):
    return pl.pallas_call(
        _tc2_body,
        grid=(N // _BM,),
        in_specs=[
            pl.BlockSpec((_BM, D_HID), lambda i: (i, 0)),
            pl.BlockSpec((NC, _BM, D_HID), lambda i: (0, i, 0)),
            pl.BlockSpec((_BM, 1), lambda i: (i, 0)),
            pl.BlockSpec((1, D_HID), lambda i: (0, 0)),
            pl.BlockSpec((D_HID, D_HID), lambda i: (0, 0)),
        ],
        out_specs=pl.BlockSpec((_BM, D_HID), lambda i: (i, 0)),
        out_shape=jax.ShapeDtypeStruct((N, D_HID), jnp.float32),
    )(u1, aggp, dinv, b1, w2)


def _tc3_body(u2_ref, aggp_ref, dinv_ref, bmu_ref, bls_ref, decw_ref, mask_ref,
              decb_ref, mu_ref, ls_ref, expr_ref):
    out2 = dinv_ref[...] * (aggp_ref[0] + aggp_ref[1] + u2_ref[...])
    mu = out2[:, :D_LAT] + bmu_ref[...]
    ls = out2[:, D_LAT:2 * D_LAT] + bls_ref[...]
    mu_ref[...] = mu
    ls_ref[...] = ls
    t = jnp.dot(mu, decw_ref[...] * mask_ref[...],
                preferred_element_type=jnp.float32) + decb_ref[...]
    t = t - jnp.max(t, axis=-1, keepdims=True)
    et = jnp.exp(t)
    expr_ref[...] = et / jnp.sum(et, axis=-1, keepdims=True)


def _tc3(u2, aggp, dinv, bmu, bls, decw, mask, decb):
    return pl.pallas_call(
        _tc3_body,
        grid=(N // _BM,),
        in_specs=[
            pl.BlockSpec((_BM, D_HID), lambda i: (i, 0)),
            pl.BlockSpec((NC, _BM, D_HID), lambda i: (0, i, 0)),
            pl.BlockSpec((_BM, 1), lambda i: (i, 0)),
            pl.BlockSpec((1, D_LAT), lambda i: (0, 0)),
            pl.BlockSpec((1, D_LAT), lambda i: (0, 0)),
            pl.BlockSpec((D_LAT, D_OUT), lambda i: (0, 0)),
            pl.BlockSpec((D_LAT, D_OUT), lambda i: (0, 0)),
            pl.BlockSpec((1, D_OUT), lambda i: (0, 0)),
        ],
        out_specs=[
            pl.BlockSpec((_BM, D_LAT), lambda i: (i, 0)),
            pl.BlockSpec((_BM, D_LAT), lambda i: (i, 0)),
            pl.BlockSpec((_BM, D_OUT), lambda i: (i, 0)),
        ],
        out_shape=[
            jax.ShapeDtypeStruct((N, D_LAT), jnp.float32),
            jax.ShapeDtypeStruct((N, D_LAT), jnp.float32),
            jax.ShapeDtypeStruct((N, D_OUT), jnp.float32),
        ],
    )(u2, aggp, dinv, bmu, bls, decw, mask, decb)


def _tc4_body(a_ref, b_ref, out_ref):
    out_ref[...] = lax.dot_general(
        a_ref[...], b_ref[...], (((1,), (1,)), ((), ())),
        preferred_element_type=jnp.float32)


_BM4 = 400


def _tc4(z):
    return pl.pallas_call(
        _tc4_body,
        grid=(N // _BM4,),
        in_specs=[
            pl.BlockSpec((_BM4, D_LAT), lambda i: (i, 0)),
            pl.BlockSpec((N, D_LAT), lambda i: (0, 0)),
        ],
        out_specs=pl.BlockSpec((_BM4, N), lambda i: (i, 0)),
        out_shape=jax.ShapeDtypeStruct((N, N), jnp.float32),
    )(z, z)


def kernel(x, edge_index, W1, b1, Wmu, bmu, Wls, bls, decW, decb, dec_mask):
    pad = E_PAD - E
    src_p = jnp.concatenate([edge_index[0], jnp.zeros((pad,), jnp.int32)])
    dst_p = jnp.concatenate([edge_index[1], jnp.full((pad,), PAD_DST, jnp.int32)])
    src2d = src_p.reshape(NW, NHALF, NCHUNK_H, C)
    dst2d = dst_p.reshape(NW, NHALF, NCHUNK_H, C)
    dst3d_deg = dst_p.reshape(NW, NCHUNK, C)
    z1 = jnp.zeros((ROWS_PER_TILE,), jnp.float32)
    z128 = jnp.zeros((ROWS_PER_TILE, D_HID), jnp.float32)

    degp = _sc_deg(dst3d_deg, z1)
    u1, dinv = _tc1(x, W1, degp.T)
    agg1p = _sc_segsum_128(u1, src2d, dst2d, z128)
    w2 = jnp.concatenate(
        [Wmu, Wls, jnp.zeros((D_HID, D_HID - 2 * D_LAT), jnp.float32)], axis=1)
    u2 = _tc2(u1, agg1p, dinv, b1.reshape(1, D_HID), w2)
    agg2p = _sc_segsum_128(u2, src2d, dst2d, z128)
    mu, logstd, expr = _tc3(
        u2, agg2p, dinv, bmu.reshape(1, D_LAT), bls.reshape(1, D_LAT),
        decW, dec_mask, decb.reshape(1, D_OUT))
    adj = _tc4(mu)
    return (adj, expr, mu, logstd)
